# baseline (speedup 1.0000x reference)
import functools

import jax
import jax.numpy as jnp
import numpy as np
from jax import lax
from jax.experimental import pallas as pl
from jax.experimental.pallas import tpu as pltpu
from jax.experimental.pallas import tpu_sc as plsc

N = 10000
E = 320000
D = 128
S = 16
SD = S * D

NC = 2
NS = 16
NW = NC * NS
E_PER_TILE = E // NW
CHUNK = 16
N_CHUNKS = E_PER_TILE // CHUNK
BLK = 400
CPB = BLK // CHUNK
ROWS_PER_TILE = 624
ZREM = N - NS * ROWS_PER_TILE

_ORDER = np.empty((32,), dtype=np.int64)
_ORDER[0::2] = np.arange(16)
_ORDER[1::2] = 16 + np.arange(16)
COLPERM = np.arange(SD).reshape(S, D // 32, 32)[..., _ORDER].reshape(-1)


def _mm_body(x_ref, w_ref, o_ref):
    y = jnp.dot(x_ref[...], w_ref[...], preferred_element_type=jnp.float32)
    ilo = lax.bitcast_convert_type(y[:, :SD // 2], jnp.int32)
    ihi = lax.bitcast_convert_type(y[:, SD // 2:], jnp.int32)
    lo16 = lax.shift_right_logical(ilo + jnp.int32(0x8000), 16)
    hi16 = (ihi + jnp.int32(0x8000)) & jnp.int32(-65536)
    o_ref[...] = lo16 | hi16


def _xw(x, wcat):
    mb = 400
    return pl.pallas_call(
        _mm_body,
        grid=(N // mb,),
        in_specs=[pl.BlockSpec((mb, D), lambda i: (i, 0)),
                  pl.BlockSpec((D, SD), lambda i: (0, 0))],
        out_specs=pl.BlockSpec((mb, SD // 2), lambda i: (i, 0)),
        out_shape=jax.ShapeDtypeStruct((N, SD // 2), jnp.int32),
    )(x, wcat)


def _bcast_lane(vec, lane):
    idx = jnp.full((CHUNK,), lane, jnp.int32)
    return vec.at[idx].get(mode="promise_in_bounds")


def _sc_body(xw_hbm, src_hbm, dst_hbm, attr_hbm, out_hbm,
             src_blk, dst_blk, attr_blk, dst_va, dst_vb, src_va, src_vb,
             rows_a, rows_b, msg_a, msg_b, zbuf_v, acc_sh,
             sem_a, sem_b, ssem_a, ssem_b):
    cid = lax.axis_index("c")
    sid = lax.axis_index("s")
    wid = cid * NS + sid
    base = wid * E_PER_TILE

    for r in range(16):
        for c8 in range(D // 16):
            zbuf_v[r, pl.ds(c8 * 16, 16)] = jnp.zeros((16,), jnp.float32)
    for k in range(ROWS_PER_TILE // 16):
        pltpu.sync_copy(
            zbuf_v, acc_sh.at[pl.ds(sid * ROWS_PER_TILE + k * 16, 16)])

    @pl.when(sid == NS - 1)
    def _zero_rem():
        pltpu.sync_copy(zbuf_v, acc_sh.at[pl.ds(NS * ROWS_PER_TILE, ZREM)])

    plsc.subcore_barrier()

    def stage_block(b):
        eb = base + b * BLK
        pltpu.sync_copy(src_hbm.at[pl.ds(eb, BLK)], src_blk)
        pltpu.sync_copy(dst_hbm.at[pl.ds(eb, BLK)], dst_blk)
        pltpu.sync_copy(attr_hbm.at[pl.ds(4 * eb, 4 * BLK)], attr_blk)

    def start_gather(off, src_ref, rows_ref, sem):
        src_ref[...] = src_blk[pl.ds(off, CHUNK)]
        pltpu.async_copy(xw_hbm.at[src_ref], rows_ref, sem)

    def do_chunk(j, src_cur, rows_cur, sem_cur, src_nxt, rows_nxt, sem_nxt,
                 dst_cur, msg_cur, ssem_cur):
        off = (j % CPB) * CHUNK

        @pl.when(j >= 2)
        def _drain_scatter():
            pltpu.make_async_copy(msg_cur, acc_sh.at[dst_cur],
                                  ssem_cur).wait()

        dst_cur[...] = dst_blk[pl.ds(off, CHUNK)]
        a = [attr_blk[pl.ds(4 * off + 16 * t, CHUNK)] for t in range(4)]
        lanes = lax.iota(jnp.int32, CHUNK)
        quarter = lanes >> 2
        u = []
        for d in range(4):
            idx = ((lanes & 3) << 2) + d
            g = [a[t].at[idx].get(mode="promise_in_bounds") for t in range(4)]
            u.append(jnp.where(quarter == 0, g[0],
                               jnp.where(quarter == 1, g[1],
                                         jnp.where(quarter == 2, g[2], g[3]))))
        c0 = 1.0 - u[0]
        c1 = 1.0 - u[1]
        c2 = 1.0 - u[2]
        c3 = 1.0 - u[3]
        pq = (c0 * c1, u[0] * c1, c0 * u[1], u[0] * u[1],
              c2 * c3, u[2] * c3, c2 * u[3], u[2] * u[3])
        bvec = [pq[s & 3] * pq[4 + (s >> 2)] for s in range(S)]

        jn = j + 1

        @pl.when(jnp.logical_and(jn % CPB == 0, jn < N_CHUNKS))
        def _stage_next():
            stage_block(jn // CPB)

        @pl.when(jn < N_CHUNKS)
        def _prefetch_next():
            start_gather((jn % CPB) * CHUNK, src_nxt, rows_nxt, sem_nxt)

        pltpu.make_async_copy(xw_hbm.at[src_cur], rows_cur, sem_cur).wait()

        @plsc.parallel_loop(0, CHUNK, step=1, unroll=4)
        def edge_body(e):
            accs = [jnp.zeros((CHUNK,), jnp.float32) for _ in range(D // 16)]
            for s in range(S):
                bs = _bcast_lane(bvec[s], e)
                for g in range(D // 32):
                    w = rows_cur[e, pl.ds(s * 64 + g * 16, 16)]
                    lo = lax.bitcast_convert_type(w << 16, jnp.float32)
                    hi = lax.bitcast_convert_type(w, jnp.float32)
                    accs[2 * g] = accs[2 * g] + bs * lo
                    accs[2 * g + 1] = accs[2 * g + 1] + bs * hi
            for v in range(D // 16):
                msg_cur[e, pl.ds(v * 16, 16)] = accs[v]
        pltpu.async_copy(msg_cur, acc_sh.at[dst_cur], ssem_cur, add=True)

    stage_block(0)
    start_gather(0, src_va, rows_a, sem_a)

    def chunk_body(j, carry):
        @pl.when(j % 2 == 0)
        def _even():
            do_chunk(j, src_va, rows_a, sem_a, src_vb, rows_b, sem_b,
                     dst_va, msg_a, ssem_a)

        @pl.when(j % 2 == 1)
        def _odd():
            do_chunk(j, src_vb, rows_b, sem_b, src_va, rows_a, sem_a,
                     dst_vb, msg_b, ssem_b)
        return carry
    lax.fori_loop(0, N_CHUNKS, chunk_body, 0)
    pltpu.make_async_copy(msg_a, acc_sh.at[dst_va], ssem_a).wait()
    pltpu.make_async_copy(msg_b, acc_sh.at[dst_vb], ssem_b).wait()
    plsc.subcore_barrier()

    r0 = sid * ROWS_PER_TILE
    pltpu.sync_copy(acc_sh.at[pl.ds(r0, ROWS_PER_TILE)],
                    out_hbm.at[cid, pl.ds(r0, ROWS_PER_TILE)])

    @pl.when(sid == NS - 1)
    def _write_rem():
        pltpu.sync_copy(acc_sh.at[pl.ds(NS * ROWS_PER_TILE, ZREM)],
                        out_hbm.at[cid, pl.ds(NS * ROWS_PER_TILE, ZREM)])


_sc_kernel = functools.partial(
    pl.kernel,
    out_type=jax.ShapeDtypeStruct((NC, N, D), jnp.float32),
    mesh=plsc.VectorSubcoreMesh(core_axis_name="c", subcore_axis_name="s"),
    scratch_types=[
        pltpu.VMEM((BLK,), jnp.int32),
        pltpu.VMEM((BLK,), jnp.int32),
        pltpu.VMEM((4 * BLK,), jnp.float32),
        pltpu.VMEM((CHUNK,), jnp.int32),
        pltpu.VMEM((CHUNK,), jnp.int32),
        pltpu.VMEM((CHUNK,), jnp.int32),
        pltpu.VMEM((CHUNK,), jnp.int32),
        pltpu.VMEM((CHUNK, SD // 2), jnp.int32),
        pltpu.VMEM((CHUNK, SD // 2), jnp.int32),
        pltpu.VMEM((CHUNK, D), jnp.float32),
        pltpu.VMEM((CHUNK, D), jnp.float32),
        pltpu.VMEM((16, D), jnp.float32),
        pltpu.VMEM_SHARED((N, D), jnp.float32),
        pltpu.SemaphoreType.DMA,
        pltpu.SemaphoreType.DMA,
        pltpu.SemaphoreType.DMA,
        pltpu.SemaphoreType.DMA,
    ],
)(_sc_body)


def _bn_body(p_ref, g_ref, b_ref, o_ref):
    s = p_ref[0] + p_ref[1]
    mean = jnp.mean(s, axis=0, keepdims=True)
    d = s - mean
    var = jnp.mean(d * d, axis=0, keepdims=True)
    xh = d * lax.rsqrt(var + 1e-5)
    y = g_ref[...] * xh + b_ref[...]
    o_ref[...] = jnp.where(y > 0, y, jnp.exp(y) - 1.0)


def _bn_elu(partials, gamma, beta):
    return pl.pallas_call(
        _bn_body,
        in_specs=[pl.BlockSpec((NC, N, D), lambda: (0, 0, 0)),
                  pl.BlockSpec((1, D), lambda: (0, 0)),
                  pl.BlockSpec((1, D), lambda: (0, 0))],
        out_specs=pl.BlockSpec((N, D), lambda: (0, 0)),
        out_shape=jax.ShapeDtypeStruct((N, D), jnp.float32),
    )(partials, gamma, beta)


def kernel(x, edge_index, edge_attr, W, gamma, beta):
    wc = jnp.transpose(W, (1, 0, 2)).reshape(D, SD)
    wcat2 = jnp.concatenate([wc[:, COLPERM[0::2]], wc[:, COLPERM[1::2]]],
                            axis=1)
    xw_i32 = _xw(x.astype(jnp.bfloat16), wcat2.astype(jnp.bfloat16))
    partials = _sc_kernel(xw_i32, edge_index[0], edge_index[1],
                          edge_attr.reshape(-1))
    return _bn_elu(partials, gamma.reshape(1, D), beta.reshape(1, D))


# speedup vs baseline: 1.0031x; 1.0031x over previous
import functools

import jax
import jax.numpy as jnp
import numpy as np
from jax import lax
from jax.experimental import pallas as pl
from jax.experimental.pallas import tpu as pltpu
from jax.experimental.pallas import tpu_sc as plsc

N = 10000
E = 320000
D = 128
S = 16
SD = S * D

NC = 2
NS = 16
NW = NC * NS
E_PER_TILE = E // NW
CHUNK = 16
N_CHUNKS = E_PER_TILE // CHUNK
BLK = 400
CPB = BLK // CHUNK
ROWS_PER_TILE = 624
ZREM = N - NS * ROWS_PER_TILE

_ORDER = np.empty((32,), dtype=np.int64)
_ORDER[0::2] = np.arange(16)
_ORDER[1::2] = 16 + np.arange(16)
COLPERM = np.arange(SD).reshape(S, D // 32, 32)[..., _ORDER].reshape(-1)


def _mm_body(x_ref, w_ref, o_ref):
    y = jnp.dot(x_ref[...].astype(jnp.bfloat16),
                w_ref[...].astype(jnp.bfloat16),
                preferred_element_type=jnp.float32)
    ilo = lax.bitcast_convert_type(y[:, :SD // 2], jnp.int32)
    ihi = lax.bitcast_convert_type(y[:, SD // 2:], jnp.int32)
    lo16 = lax.shift_right_logical(ilo + jnp.int32(0x8000), 16)
    hi16 = (ihi + jnp.int32(0x8000)) & jnp.int32(-65536)
    o_ref[...] = lo16 | hi16


def _xw(x, wcat):
    mb = 400
    return pl.pallas_call(
        _mm_body,
        grid=(N // mb,),
        in_specs=[pl.BlockSpec((mb, D), lambda i: (i, 0)),
                  pl.BlockSpec((D, SD), lambda i: (0, 0))],
        out_specs=pl.BlockSpec((mb, SD // 2), lambda i: (i, 0)),
        out_shape=jax.ShapeDtypeStruct((N, SD // 2), jnp.int32),
    )(x, wcat)


def _bcast_lane(vec, lane):
    idx = jnp.full((CHUNK,), lane, jnp.int32)
    return vec.at[idx].get(mode="promise_in_bounds")


def _sc_body(xw_hbm, src_hbm, dst_hbm, attr_hbm, out_hbm,
             src_blk, dst_blk, attr_blk, dst_va, dst_vb, src_va, src_vb,
             rows_a, rows_b, msg_a, msg_b, zbuf_v, acc_sh,
             sem_a, sem_b, ssem_a, ssem_b):
    cid = lax.axis_index("c")
    sid = lax.axis_index("s")
    wid = cid * NS + sid
    base = wid * E_PER_TILE

    for r in range(16):
        for c8 in range(D // 16):
            zbuf_v[r, pl.ds(c8 * 16, 16)] = jnp.zeros((16,), jnp.float32)
    for k in range(ROWS_PER_TILE // 16):
        pltpu.sync_copy(
            zbuf_v, acc_sh.at[pl.ds(sid * ROWS_PER_TILE + k * 16, 16)])

    @pl.when(sid == NS - 1)
    def _zero_rem():
        pltpu.sync_copy(zbuf_v, acc_sh.at[pl.ds(NS * ROWS_PER_TILE, ZREM)])

    plsc.subcore_barrier()

    def stage_block(b):
        eb = base + b * BLK
        pltpu.sync_copy(src_hbm.at[pl.ds(eb, BLK)], src_blk)
        pltpu.sync_copy(dst_hbm.at[pl.ds(eb, BLK)], dst_blk)
        pltpu.sync_copy(attr_hbm.at[pl.ds(4 * eb, 4 * BLK)], attr_blk)

    def start_gather(off, src_ref, rows_ref, sem):
        src_ref[...] = src_blk[pl.ds(off, CHUNK)]
        pltpu.async_copy(xw_hbm.at[src_ref], rows_ref, sem)

    def do_chunk(j, src_cur, rows_cur, sem_cur, src_nxt, rows_nxt, sem_nxt,
                 dst_cur, msg_cur, ssem_cur):
        off = (j % CPB) * CHUNK

        @pl.when(j >= 2)
        def _drain_scatter():
            pltpu.make_async_copy(msg_cur, acc_sh.at[dst_cur],
                                  ssem_cur).wait()

        dst_cur[...] = dst_blk[pl.ds(off, CHUNK)]
        a = [attr_blk[pl.ds(4 * off + 16 * t, CHUNK)] for t in range(4)]
        lanes = lax.iota(jnp.int32, CHUNK)
        quarter = lanes >> 2
        u = []
        for d in range(4):
            idx = ((lanes & 3) << 2) + d
            g = [a[t].at[idx].get(mode="promise_in_bounds") for t in range(4)]
            u.append(jnp.where(quarter == 0, g[0],
                               jnp.where(quarter == 1, g[1],
                                         jnp.where(quarter == 2, g[2], g[3]))))
        c0 = 1.0 - u[0]
        c1 = 1.0 - u[1]
        c2 = 1.0 - u[2]
        c3 = 1.0 - u[3]
        pq = (c0 * c1, u[0] * c1, c0 * u[1], u[0] * u[1],
              c2 * c3, u[2] * c3, c2 * u[3], u[2] * u[3])
        bvec = [pq[s & 3] * pq[4 + (s >> 2)] for s in range(S)]

        jn = j + 1

        @pl.when(jnp.logical_and(jn % CPB == 0, jn < N_CHUNKS))
        def _stage_next():
            stage_block(jn // CPB)

        @pl.when(jn < N_CHUNKS)
        def _prefetch_next():
            start_gather((jn % CPB) * CHUNK, src_nxt, rows_nxt, sem_nxt)

        pltpu.make_async_copy(xw_hbm.at[src_cur], rows_cur, sem_cur).wait()

        @plsc.parallel_loop(0, CHUNK, step=1, unroll=4)
        def edge_body(e):
            accs = [jnp.zeros((CHUNK,), jnp.float32) for _ in range(D // 16)]
            for s in range(S):
                bs = _bcast_lane(bvec[s], e)
                for g in range(D // 32):
                    w = rows_cur[e, pl.ds(s * 64 + g * 16, 16)]
                    lo = lax.bitcast_convert_type(w << 16, jnp.float32)
                    hi = lax.bitcast_convert_type(w, jnp.float32)
                    accs[2 * g] = accs[2 * g] + bs * lo
                    accs[2 * g + 1] = accs[2 * g + 1] + bs * hi
            for v in range(D // 16):
                msg_cur[e, pl.ds(v * 16, 16)] = accs[v]
        pltpu.async_copy(msg_cur, acc_sh.at[dst_cur], ssem_cur, add=True)

    stage_block(0)
    start_gather(0, src_va, rows_a, sem_a)

    def chunk_body(j, carry):
        @pl.when(j % 2 == 0)
        def _even():
            do_chunk(j, src_va, rows_a, sem_a, src_vb, rows_b, sem_b,
                     dst_va, msg_a, ssem_a)

        @pl.when(j % 2 == 1)
        def _odd():
            do_chunk(j, src_vb, rows_b, sem_b, src_va, rows_a, sem_a,
                     dst_vb, msg_b, ssem_b)
        return carry
    lax.fori_loop(0, N_CHUNKS, chunk_body, 0)
    pltpu.make_async_copy(msg_a, acc_sh.at[dst_va], ssem_a).wait()
    pltpu.make_async_copy(msg_b, acc_sh.at[dst_vb], ssem_b).wait()
    plsc.subcore_barrier()

    r0 = sid * ROWS_PER_TILE
    pltpu.sync_copy(acc_sh.at[pl.ds(r0, ROWS_PER_TILE)],
                    out_hbm.at[cid, pl.ds(r0, ROWS_PER_TILE)])

    @pl.when(sid == NS - 1)
    def _write_rem():
        pltpu.sync_copy(acc_sh.at[pl.ds(NS * ROWS_PER_TILE, ZREM)],
                        out_hbm.at[cid, pl.ds(NS * ROWS_PER_TILE, ZREM)])


_sc_kernel = functools.partial(
    pl.kernel,
    out_type=jax.ShapeDtypeStruct((NC, N, D), jnp.float32),
    mesh=plsc.VectorSubcoreMesh(core_axis_name="c", subcore_axis_name="s"),
    scratch_types=[
        pltpu.VMEM((BLK,), jnp.int32),
        pltpu.VMEM((BLK,), jnp.int32),
        pltpu.VMEM((4 * BLK,), jnp.float32),
        pltpu.VMEM((CHUNK,), jnp.int32),
        pltpu.VMEM((CHUNK,), jnp.int32),
        pltpu.VMEM((CHUNK,), jnp.int32),
        pltpu.VMEM((CHUNK,), jnp.int32),
        pltpu.VMEM((CHUNK, SD // 2), jnp.int32),
        pltpu.VMEM((CHUNK, SD // 2), jnp.int32),
        pltpu.VMEM((CHUNK, D), jnp.float32),
        pltpu.VMEM((CHUNK, D), jnp.float32),
        pltpu.VMEM((16, D), jnp.float32),
        pltpu.VMEM_SHARED((N, D), jnp.float32),
        pltpu.SemaphoreType.DMA,
        pltpu.SemaphoreType.DMA,
        pltpu.SemaphoreType.DMA,
        pltpu.SemaphoreType.DMA,
    ],
)(_sc_body)


def _bn_body(p_ref, g_ref, b_ref, o_ref):
    s = p_ref[0] + p_ref[1]
    mean = jnp.mean(s, axis=0, keepdims=True)
    d = s - mean
    var = jnp.mean(d * d, axis=0, keepdims=True)
    xh = d * lax.rsqrt(var + 1e-5)
    y = g_ref[...] * xh + b_ref[...]
    o_ref[...] = jnp.where(y > 0, y, jnp.exp(y) - 1.0)


def _bn_elu(partials, gamma, beta):
    return pl.pallas_call(
        _bn_body,
        in_specs=[pl.BlockSpec((NC, N, D), lambda: (0, 0, 0)),
                  pl.BlockSpec((1, D), lambda: (0, 0)),
                  pl.BlockSpec((1, D), lambda: (0, 0))],
        out_specs=pl.BlockSpec((N, D), lambda: (0, 0)),
        out_shape=jax.ShapeDtypeStruct((N, D), jnp.float32),
    )(partials, gamma, beta)


def kernel(x, edge_index, edge_attr, W, gamma, beta):
    wc = jnp.transpose(W, (1, 0, 2)).reshape(D, SD)
    wcat2 = jnp.concatenate([wc[:, COLPERM[0::2]], wc[:, COLPERM[1::2]]],
                            axis=1)
    xw_i32 = _xw(x, wcat2)
    partials = _sc_kernel(xw_i32, edge_index[0], edge_index[1],
                          edge_attr.reshape(-1))
    return _bn_elu(partials, gamma.reshape(1, D), beta.reshape(1, D))
